# baseline (speedup 1.0000x reference)
import functools

import jax
import jax.numpy as jnp
from jax import lax
from jax.experimental import pallas as pl
from jax.experimental.pallas import tpu as pltpu
from jax.experimental.pallas import tpu_sc as plsc

D = 64
CHUNK = 128


@functools.lru_cache(maxsize=None)
def _make(hist, nbc, nw, nchw):
    mesh = plsc.VectorSubcoreMesh(core_axis_name="c", subcore_axis_name="s")
    nc = plsc.get_sparse_core_info().num_cores
    per_w = nchw * CHUNK

    @functools.partial(
        pl.kernel,
        mesh=mesh,
        compiler_params=pltpu.CompilerParams(
            use_tc_tiling_on_sc=False, needs_layout_passes=False
        ),
        out_type=jax.ShapeDtypeStruct((hist, D // 8, nbc, 8, CHUNK), jnp.float32),
        scratch_types=[
            pltpu.VMEM((per_w,), jnp.int32),
            pltpu.VMEM((CHUNK, D), jnp.float32),
            pltpu.VMEM((CHUNK, D), jnp.float32),
            pltpu.VMEM((D, CHUNK), jnp.float32),
            pltpu.VMEM((D, CHUNK), jnp.float32),
            pltpu.SemaphoreType.DMA,
            pltpu.SemaphoreType.DMA,
            pltpu.SemaphoreType.DMA,
            pltpu.SemaphoreType.DMA,
        ],
    )
    def k(idx_hbm, table_hbm, out_hbm, idx_v, r0, r1, t0, t1, gs0, gs1, ws0, ws1):
        wid = lax.axis_index("s") * nc + lax.axis_index("c")
        chbase = wid * nchw
        pltpu.sync_copy(idx_hbm.at[pl.ds(chbase * CHUNK, per_w)], idx_v)

        bufs = ((r0, t0, gs0, ws0), (r1, t1, gs1, ws1))
        lane = lax.iota(jnp.int32, 16)

        def fire(j, rb, gs):
            pltpu.make_async_copy(
                table_hbm.at[idx_v.at[pl.ds(j * CHUNK, CHUNK)]], rb, gs
            ).start()

        dconsts = [jnp.full((16,), d, jnp.int32) for d in range(D)]

        def transpose(rb, tb):
            @plsc.parallel_loop(0, CHUNK // 16)
            def tbody(kk):
                cbase = 16 * kk
                cv = lane + cbase
                for d in range(D):
                    vals = plsc.load_gather(rb, [cv, dconsts[d]])
                    tb[d, pl.ds(cbase, 16)] = vals

        def writes_start(t, tc, tb, ws):
            for tr in range(D // 8):
                pltpu.make_async_copy(
                    tb.at[pl.ds(8 * tr, 8)], out_hbm.at[t, tr, tc], ws
                ).start()

        def writes_drain(tb, ws):
            for tr in range(D // 8):
                pltpu.make_async_copy(
                    tb.at[pl.ds(8 * tr, 8)], out_hbm.at[0, tr, 0], ws
                ).wait()

        fire(0, r0, gs0)

        def body(i, _):
            for b in range(2):
                j = i * 2 + b
                rb, tb, gs, ws = bufs[b]
                ro, _, go, _ = bufs[1 - b]

                @pl.when(j + 1 < nchw)
                def _():
                    fire(j + 1, ro, go)

                pltpu.make_async_copy(
                    table_hbm.at[idx_v.at[pl.ds(0, CHUNK)]], rb, gs
                ).wait()

                @pl.when(j >= 2)
                def _():
                    writes_drain(tb, ws)

                transpose(rb, tb)
                plsc.subcore_barrier()
                ch = chbase + j
                t = ch // nbc
                tc = ch % nbc
                writes_start(t, tc, tb, ws)
            return 0

        lax.fori_loop(0, nchw // 2, body, 0)
        writes_drain(t0, ws0)
        writes_drain(t1, ws1)

    return k


def kernel(token_ids, weight):
    batch, hist = token_ids.shape
    total = batch * hist
    nw = 32
    assert batch % CHUNK == 0 and D % 8 == 0
    nbc = batch // CHUNK
    nch = hist * nbc
    assert nch % (2 * nw) == 0
    nchw = nch // nw
    idx = token_ids.T.reshape(total).astype(jnp.int32)
    out = _make(hist, nbc, nw, nchw)(idx, weight)
    return out.transpose(2, 4, 0, 1, 3).reshape(batch, hist, D)


# speedup vs baseline: 1.6406x; 1.6406x over previous
import functools

import jax
import jax.numpy as jnp
from jax import lax
from jax.experimental import pallas as pl
from jax.experimental.pallas import tpu as pltpu
from jax.experimental.pallas import tpu_sc as plsc

D = 64
CHUNK = 128


@functools.lru_cache(maxsize=None)
def _make(hist, nbc, nw, nchw):
    mesh = plsc.VectorSubcoreMesh(core_axis_name="c", subcore_axis_name="s")
    nc = plsc.get_sparse_core_info().num_cores
    per_w = nchw * CHUNK

    @functools.partial(
        pl.kernel,
        mesh=mesh,
        compiler_params=pltpu.CompilerParams(
            use_tc_tiling_on_sc=False, needs_layout_passes=False
        ),
        out_type=jax.ShapeDtypeStruct((hist, D // 8, nbc, 8, CHUNK), jnp.float32),
        scratch_types=[
            pltpu.VMEM((per_w,), jnp.int32),
            pltpu.VMEM((CHUNK, D), jnp.float32),
            pltpu.VMEM((CHUNK, D), jnp.float32),
            pltpu.VMEM((D, CHUNK + 1), jnp.float32),
            pltpu.VMEM((D, CHUNK + 1), jnp.float32),
            pltpu.SemaphoreType.DMA,
            pltpu.SemaphoreType.DMA,
            pltpu.SemaphoreType.DMA,
            pltpu.SemaphoreType.DMA,
        ],
    )
    def k(idx_hbm, table_hbm, out_hbm, idx_v, r0, r1, t0, t1, gs0, gs1, ws0, ws1):
        wid = lax.axis_index("s") * nc + lax.axis_index("c")
        chbase = wid * nchw
        pltpu.sync_copy(idx_hbm.at[pl.ds(chbase * CHUNK, per_w)], idx_v)

        bufs = ((r0, t0, gs0, ws0), (r1, t1, gs1, ws1))
        lane = lax.iota(jnp.int32, 16)

        def fire(j, rb, gs):
            pltpu.make_async_copy(
                table_hbm.at[idx_v.at[pl.ds(j * CHUNK, CHUNK)]], rb, gs
            ).start()

        dvecs = [lane + d0 for d0 in range(0, D, 16)]

        def transpose(rb, tb):
            @plsc.parallel_loop(0, CHUNK)
            def tbody(c):
                cv = jnp.full((16,), 0, jnp.int32) + c
                for kb in range(D // 16):
                    vals = rb[c, pl.ds(16 * kb, 16)]
                    plsc.store_scatter(tb, [dvecs[kb], cv], vals)

        def writes_start(t, tc, tb, ws):
            for tr in range(D // 8):
                pltpu.make_async_copy(
                    tb.at[pl.ds(8 * tr, 8), pl.ds(0, CHUNK)],
                    out_hbm.at[t, tr, tc],
                    ws,
                ).start()

        def writes_drain(tb, ws):
            for tr in range(D // 8):
                pltpu.make_async_copy(
                    tb.at[pl.ds(8 * tr, 8), pl.ds(0, CHUNK)],
                    out_hbm.at[0, tr, 0],
                    ws,
                ).wait()

        fire(0, r0, gs0)

        def body(i, _):
            for b in range(2):
                j = i * 2 + b
                rb, tb, gs, ws = bufs[b]
                ro, _, go, _ = bufs[1 - b]

                @pl.when(j + 1 < nchw)
                def _():
                    fire(j + 1, ro, go)

                pltpu.make_async_copy(
                    table_hbm.at[idx_v.at[pl.ds(0, CHUNK)]], rb, gs
                ).wait()

                @pl.when(j >= 2)
                def _():
                    writes_drain(tb, ws)

                transpose(rb, tb)
                plsc.subcore_barrier()
                ch = chbase + j
                t = ch // nbc
                tc = ch % nbc
                writes_start(t, tc, tb, ws)
            return 0

        lax.fori_loop(0, nchw // 2, body, 0)
        writes_drain(t0, ws0)
        writes_drain(t1, ws1)

    return k


def kernel(token_ids, weight):
    batch, hist = token_ids.shape
    total = batch * hist
    nw = 32
    assert batch % CHUNK == 0 and D % 8 == 0
    nbc = batch // CHUNK
    nch = hist * nbc
    assert nch % (2 * nw) == 0
    nchw = nch // nw
    idx = token_ids.T.reshape(total).astype(jnp.int32)
    out = _make(hist, nbc, nw, nchw)(idx, weight)
    return out.transpose(2, 4, 0, 1, 3).reshape(batch, hist, D)
